# Initial kernel scaffold; baseline (speedup 1.0000x reference)
import jax
import jax.numpy as jnp
from jax.experimental import pallas as pl


def kernel(x, params, edge_index, batch):
    raise NotImplementedError("write your pallas kernel here")


# speedup vs baseline: 2.8760x; 2.8760x over previous
import functools

import jax
import jax.numpy as jnp
from jax import lax
from jax.experimental import pallas as pl
from jax.experimental.pallas import tpu as pltpu
from jax.experimental.pallas import tpu_sc as plsc

NUM_GRAPHS = 8
HID = 64
K_SUPER = 16
LATENT = 32

TW = 144
CE = 128
NSC = 2
NSUB = 16
NW = NSC * NSUB

F32 = jnp.float32


def _sp(x):
    return jnp.maximum(x, 0.0) + jnp.log1p(jnp.exp(-jnp.abs(x)))


def _lnk(x, g, b):
    m = jnp.mean(x, axis=-1, keepdims=True)
    v = jnp.mean((x - m) ** 2, axis=-1, keepdims=True)
    return (x - m) / jnp.sqrt(v + 1e-5) * g + b


def _lnk_mxu(x, g, b):
    u = jnp.full((x.shape[1], x.shape[1]), 1.0 / x.shape[1], F32)
    m = jnp.dot(x, u, preferred_element_type=F32)
    d = x - m
    v = jnp.dot(d * d, u, preferred_element_type=F32)
    return d * lax.rsqrt(v + 1e-5) * g + b


def _prep_body(xr, wd, ws, td, ts):
    xb = xr[...]
    pv = xb[:, 0:4]
    z = jnp.zeros((xb.shape[0], 12), F32)
    td[...] = jnp.concatenate(
        [jnp.dot(xb, wd[...], preferred_element_type=F32), -pv, z], axis=1)
    ts[...] = jnp.concatenate(
        [jnp.dot(xb, ws[...], preferred_element_type=F32), pv, z], axis=1)


def _edge_body(g, gw, vecp, w2e, w3e, vw, oh, om):
    gb = g[...]
    geb = gb[:, 0:64]
    gvb = gb[:, 64:128]
    px = gb[:, 128:129]
    py = gb[:, 129:130]
    vx = gb[:, 130:131]
    vy = gb[:, 131:132]
    dist = px * px + py * py
    dvr = vx * px + vy * py
    r2 = jnp.minimum(1.0 / (dist + 0.05), 20.0)
    r6 = jnp.minimum(r2 * r2 * r2, 400.0)
    r12 = jnp.minimum(r6 * r6, 160000.0)
    zero = jnp.zeros_like(px)
    geo = jnp.concatenate([dist, dvr, r2, r6, r12, zero, zero, zero], axis=1)
    gpro = jnp.dot(geo, gw[...], preferred_element_type=F32)
    vp = vecp[...]
    e1 = _sp(_lnk_mxu(geb + gpro[:, 0:64] + vp[0:1, :], vp[1:2, :], vp[2:3, :]))
    e2 = _sp(jnp.dot(e1, w2e[...], preferred_element_type=F32) + vp[3:4, :])
    mh = jnp.dot(e2, w3e[...], preferred_element_type=F32) + vp[4:5, :]
    v1 = _sp(_lnk_mxu(gvb + gpro[:, 64:128] + vp[5:6, :], vp[6:7, :], vp[7:8, :]))
    sv = jnp.dot(v1, vw[...], preferred_element_type=F32)[:, 0:1] + vp[8:9, 0:1]
    oh[...] = mh
    om[...] = jnp.concatenate(
        [sv * px, sv * py, jnp.ones_like(px)] + [zero] * 13, axis=1)


def _node_update(xb, ah, am, w1hx, w1hm, nv, w2h):
    mvx = am[:, 0:1]
    mvy = am[:, 1:2]
    cnt = am[:, 2:3]
    den = jnp.maximum(cnt, 1.0)
    mh = ah / den
    mvx = mvx / den
    mvy = mvy / den
    mvn = jnp.sqrt((mvx + 1e-8) ** 2 + (mvy + 1e-8) ** 2)
    t = (jnp.dot(xb, w1hx[...], preferred_element_type=F32)
         + jnp.dot(mh, w1hm[...], preferred_element_type=F32)
         + mvn * nv[6:7, :] + nv[0:1, :])
    hh = _sp(_lnk(t, nv[1:2, :], nv[2:3, :]))
    return jnp.dot(hh, w2h[...], preferred_element_type=F32) + nv[3:4, :]


def _node1_body(xr, a0h, a1h, a0m, a1m, w1hx, w1hm, nvec, w2h, wsc,
                wd2, ws2, h1, t2d, t2s):
    xb = xr[...]
    ah = a0h[...] + a1h[...]
    am = a0m[...] + a1m[...]
    nv = nvec[...]
    upd = _node_update(xb, ah, am, w1hx, w1hm, nv, w2h)
    short = jnp.dot(xb, wsc[...], preferred_element_type=F32) + nv[7:8, :]
    h1b = _lnk(jnp.maximum(short + upd, 0.0), nv[4:5, :], nv[5:6, :])
    h1[...] = h1b
    pv = xb[:, 0:4]
    z = jnp.zeros((xb.shape[0], 12), F32)
    t2d[...] = jnp.concatenate(
        [jnp.dot(h1b, wd2[...], preferred_element_type=F32), -pv, z], axis=1)
    t2s[...] = jnp.concatenate(
        [jnp.dot(h1b, ws2[...], preferred_element_type=F32), pv, z], axis=1)


def _final1_body(h1r, xpvr, btr, a0h, a1h, a0m, a1m,
                 w1hx, w1hm, nvec, w2h, wp, fv16,
                 s_out, r_acc, ent_acc):
    i = pl.program_id(0)
    h1b = h1r[...]
    ah = a0h[...] + a1h[...]
    am = a0m[...] + a1m[...]
    nv = nvec[...]
    upd = _node_update(h1b, ah, am, w1hx, w1hm, nv, w2h)
    h2 = _lnk(jnp.maximum(h1b + upd, 0.0), nv[4:5, :], nv[5:6, :])

    lg = jnp.dot(h2, wp[...], preferred_element_type=F32) + fv16[0:1, :]
    mx = jnp.max(lg, axis=-1, keepdims=True)
    ex = jnp.exp(lg - mx)
    s = ex / jnp.sum(ex, axis=-1, keepdims=True)
    s_out[...] = s

    n = h1b.shape[0]
    ent = s * jnp.log(s + 1e-8)

    xpv = xpvr[...]
    one = jnp.ones((n, 1), F32)
    zed = jnp.zeros((n, 5), F32)
    hx = jnp.concatenate([h2, xpv[:, 0:1], xpv[:, 1:2], one, zed], axis=1)
    bt = btr[...]
    iota8 = lax.broadcasted_iota(jnp.int32, (n, NUM_GRAPHS), 1)
    oneh = (bt == iota8).astype(F32)
    rows = []
    for g in range(NUM_GRAPHS):
        sg = s * oneh[:, g:g + 1]
        rows.append(lax.dot_general(sg, hx, (((0,), (0,)), ((), ())),
                                    preferred_element_type=F32))
    r = jnp.concatenate(rows, axis=0)
    ent_s = jnp.sum(ent).reshape(1, 1)

    @pl.when(i == 0)
    def _():
        r_acc[...] = r
        ent_acc[...] = ent_s

    @pl.when(i > 0)
    def _():
        r_acc[...] = r_acc[...] + r
        ent_acc[...] = ent_acc[...] + ent_s


def _final2_body(n, rr, wo1, wo2, fv64, fv32, entr, latf, muf, loss):
    r = rr[...]
    den = r[:, 66:67] + 1e-8
    pooled = r[:, 0:64] / den
    mux = r[:, 64:65] / den
    muy = r[:, 65:66] / den
    z1 = jnp.maximum(jnp.dot(pooled, wo1[...], preferred_element_type=F32)
                     + fv64[0:1, :], 0.0)
    lat = (jnp.dot(z1, wo2[...], preferred_element_type=F32)
           + fv32[0:1, :]) * fv32[1:2, :]
    m = jnp.mean(lat, axis=-1, keepdims=True)
    v = jnp.mean((lat - m) ** 2, axis=-1, keepdims=True)
    latf[...] = (lat - m) / jnp.sqrt(v + 1e-5)
    z6 = jnp.zeros((r.shape[0], 6), F32)
    muf[...] = jnp.concatenate([mux, muy, z6], axis=1)
    loss[...] = -entr[...] / n


def _gather_sc_body(nch, ni, td, ts, dsti, srci, out,
                    di_v, si_v, rd_v, rs_v, sem1, sem2):
    cid = lax.axis_index("c")
    sid = lax.axis_index("s")
    wid = sid * NSC + cid

    def step(i, carry):
        c = i * NW + wid

        @pl.when(c < nch)
        def _():
            pltpu.sync_copy(dsti.at[c], di_v)
            pltpu.sync_copy(srci.at[c], si_v)
            cp1 = pltpu.async_copy(td.at[di_v], rd_v, sem1)
            cp2 = pltpu.async_copy(ts.at[si_v], rs_v, sem2)
            cp1.wait()
            cp2.wait()

            def addrow(r, carry2):
                for k in range(TW // 16):
                    rd_v[r, pl.ds(k * 16, 16)] = (
                        rd_v[r, pl.ds(k * 16, 16)] + rs_v[r, pl.ds(k * 16, 16)])
                return carry2

            lax.fori_loop(0, CE, addrow, 0)
            pltpu.sync_copy(rd_v, out.at[pl.ds(c * CE, CE)])

        return carry

    lax.fori_loop(0, ni, step, 0)


def _scatter_sc_body(n, nch, ni, eoh, eom, dsti, z64, z16, out_h, out_m,
                     acc_h, acc_m, di_v, rh_v, rm_v):
    cid = lax.axis_index("c")
    sid = lax.axis_index("s")
    wid = sid * NSC + cid
    rpt = n // NSUB

    pltpu.sync_copy(z64, acc_h.at[pl.ds(sid * rpt, rpt)])
    pltpu.sync_copy(z16, acc_m.at[pl.ds(sid * rpt, rpt)])
    plsc.subcore_barrier()

    def step(i, carry):
        c = i * NW + wid

        @pl.when(c < nch)
        def _():
            pltpu.sync_copy(dsti.at[c], di_v)
            pltpu.sync_copy(eoh.at[pl.ds(c * CE, CE)], rh_v)
            pltpu.sync_copy(eom.at[pl.ds(c * CE, CE)], rm_v)
            pltpu.sync_copy(rh_v, acc_h.at[di_v], add=True)
            pltpu.sync_copy(rm_v, acc_m.at[di_v], add=True)

        return carry

    lax.fori_loop(0, ni, step, 0)
    plsc.subcore_barrier()
    pltpu.sync_copy(acc_h.at[pl.ds(sid * rpt, rpt)],
                    out_h.at[cid, pl.ds(sid * rpt, rpt)])
    pltpu.sync_copy(acc_m.at[pl.ds(sid * rpt, rpt)],
                    out_m.at[cid, pl.ds(sid * rpt, rpt)])


def _sc_mesh():
    return plsc.VectorSubcoreMesh(core_axis_name="c", subcore_axis_name="s",
                                  num_cores=NSC, num_subcores=NSUB)


def _sc_gather(td, ts, dsti2, srci2):
    n = td.shape[0]
    nch = dsti2.shape[0]
    e = nch * CE
    ni = (nch + NW - 1) // NW
    call = pl.kernel(
        functools.partial(_gather_sc_body, nch, ni),
        out_type=jax.ShapeDtypeStruct((e, TW), F32),
        mesh=_sc_mesh(),
        scratch_types=[
            pltpu.VMEM((CE,), jnp.int32),
            pltpu.VMEM((CE,), jnp.int32),
            pltpu.VMEM((CE, TW), F32),
            pltpu.VMEM((CE, TW), F32),
            pltpu.SemaphoreType.DMA,
            pltpu.SemaphoreType.DMA,
        ],
        compiler_params=pltpu.CompilerParams(use_tc_tiling_on_sc=False),
        name="gnn_sc_gather",
    )
    del n
    return call(td, ts, dsti2, srci2)


def _sc_scatter(n, eoh, eom, dsti2, z64, z16):
    nch = dsti2.shape[0]
    ni = (nch + NW - 1) // NW
    call = pl.kernel(
        functools.partial(_scatter_sc_body, n, nch, ni),
        out_type=(jax.ShapeDtypeStruct((NSC, n, 64), F32),
                  jax.ShapeDtypeStruct((NSC, n, 16), F32)),
        mesh=_sc_mesh(),
        scratch_types=[
            pltpu.VMEM_SHARED((n, 64), F32),
            pltpu.VMEM_SHARED((n, 16), F32),
            pltpu.VMEM((CE,), jnp.int32),
            pltpu.VMEM((CE, 64), F32),
            pltpu.VMEM((CE, 16), F32),
        ],
        compiler_params=pltpu.CompilerParams(use_tc_tiling_on_sc=False),
        name="gnn_sc_scatter",
    )
    return call(eoh, eom, dsti2, z64, z16)


def _pack_edge(p, fin):
    w1e = p["phi_e"]["l1"]["W"]
    w1v = p["phi_v"]["l1"]["W"]
    wd = jnp.concatenate([w1e[:fin], w1v[:fin]], axis=1)
    ws = jnp.concatenate([w1e[fin:2 * fin], w1v[fin:2 * fin]], axis=1)
    gw = jnp.pad(jnp.concatenate([w1e[2 * fin:], w1v[2 * fin:]], axis=1),
                 ((0, 3), (0, 0)))
    rows = [p["phi_e"]["l1"]["b"], p["phi_e"]["g"], p["phi_e"]["be"],
            p["phi_e"]["l2"]["b"], p["phi_e"]["l3"]["b"],
            p["phi_v"]["l1"]["b"], p["phi_v"]["g"], p["phi_v"]["be"],
            jnp.full((HID,), p["phi_v"]["l2"]["b"][0])]
    vecp = jnp.pad(jnp.stack(rows), ((0, 16 - len(rows)), (0, 0)))
    vw = jnp.pad(p["phi_v"]["l2"]["W"], ((0, 0), (0, 7)))
    return wd, ws, gw, vecp, p["phi_e"]["l2"]["W"], p["phi_e"]["l3"]["W"], vw


def _pack_node(p, ln, fin):
    w1h = p["phi_h"]["l1"]["W"]
    w1hx = w1h[:fin]
    w1hm = w1h[fin:fin + HID]
    w1hn = w1h[fin + HID]
    bsc = p["sc"]["b"] if "sc" in p else jnp.zeros((HID,), F32)
    rows = [p["phi_h"]["l1"]["b"], p["phi_h"]["g"], p["phi_h"]["be"],
            p["phi_h"]["l2"]["b"], ln["g"], ln["b"], w1hn, bsc]
    nvec = jnp.stack(rows)
    return w1hx, w1hm, nvec, p["phi_h"]["l2"]["W"]


def kernel(x, params, edge_index, batch):
    n, nf = x.shape
    e = edge_index.shape[1]
    assert e % CE == 0 and n % NSUB == 0
    nch = e // CE
    bn = 2000 if n % 2000 == 0 else n
    be = 2000 if e % 2000 == 0 else CE

    dsti2 = edge_index[1].reshape(nch, CE)
    srci2 = edge_index[0].reshape(nch, CE)
    z64 = jnp.zeros((n // NSUB, 64), F32)
    z16 = jnp.zeros((n // NSUB, 16), F32)
    xpv = x[:, 0:4]
    bt = batch.reshape(n, 1)

    p1 = params["gnn1"]
    p2 = params["gnn2"]
    wd1, ws1, gw1, vecp1, w2e1, w3e1, vw1 = _pack_edge(p1, nf)
    wd2, ws2, gw2, vecp2, w2e2, w3e2, vw2 = _pack_edge(p2, HID)
    w1hx1, w1hm1, nvec1, w2h1 = _pack_node(p1, params["ln1"], nf)
    w1hx2, w1hm2, nvec2, w2h2 = _pack_node(p2, params["ln2"], HID)
    wsc1 = p1["sc"]["W"]

    full = lambda shape: pl.BlockSpec(shape, lambda i: (0, 0))

    t1d, t1s = pl.pallas_call(
        _prep_body,
        grid=(n // bn,),
        in_specs=[pl.BlockSpec((bn, nf), lambda i: (i, 0)),
                  full((nf, 128)), full((nf, 128))],
        out_specs=[pl.BlockSpec((bn, TW), lambda i: (i, 0)),
                   pl.BlockSpec((bn, TW), lambda i: (i, 0))],
        out_shape=[jax.ShapeDtypeStruct((n, TW), F32),
                   jax.ShapeDtypeStruct((n, TW), F32)],
        name="gnn_prep",
    )(x, wd1, ws1)

    def edge_call(g, gw, vecp, w2e, w3e, vw):
        return pl.pallas_call(
            _edge_body,
            grid=(e // be,),
            in_specs=[pl.BlockSpec((be, TW), lambda i: (i, 0)),
                      full((8, 128)), full((16, 64)), full((64, 64)),
                      full((64, 64)), full((64, 8))],
            out_specs=[pl.BlockSpec((be, 64), lambda i: (i, 0)),
                       pl.BlockSpec((be, 16), lambda i: (i, 0))],
            out_shape=[jax.ShapeDtypeStruct((e, 64), F32),
                       jax.ShapeDtypeStruct((e, 16), F32)],
            name="gnn_edge_mlp",
        )(g, gw, vecp, w2e, w3e, vw)

    g1 = _sc_gather(t1d, t1s, dsti2, srci2)
    eoh1, eom1 = edge_call(g1, gw1, vecp1, w2e1, w3e1, vw1)
    acch1, accm1 = _sc_scatter(n, eoh1, eom1, dsti2, z64, z16)

    h1, t2d, t2s = pl.pallas_call(
        _node1_body,
        grid=(n // bn,),
        in_specs=[pl.BlockSpec((bn, nf), lambda i: (i, 0)),
                  pl.BlockSpec((bn, 64), lambda i: (i, 0)),
                  pl.BlockSpec((bn, 64), lambda i: (i, 0)),
                  pl.BlockSpec((bn, 16), lambda i: (i, 0)),
                  pl.BlockSpec((bn, 16), lambda i: (i, 0)),
                  full((nf, 64)), full((64, 64)), full((8, 64)),
                  full((64, 64)), full((nf, 64)),
                  full((64, 128)), full((64, 128))],
        out_specs=[pl.BlockSpec((bn, 64), lambda i: (i, 0)),
                   pl.BlockSpec((bn, TW), lambda i: (i, 0)),
                   pl.BlockSpec((bn, TW), lambda i: (i, 0))],
        out_shape=[jax.ShapeDtypeStruct((n, 64), F32),
                   jax.ShapeDtypeStruct((n, TW), F32),
                   jax.ShapeDtypeStruct((n, TW), F32)],
        name="gnn_node1",
    )(x, acch1[0], acch1[1], accm1[0], accm1[1],
      w1hx1, w1hm1, nvec1, w2h1, wsc1, wd2, ws2)

    g2 = _sc_gather(t2d, t2s, dsti2, srci2)
    eoh2, eom2 = edge_call(g2, gw2, vecp2, w2e2, w3e2, vw2)
    acch2, accm2 = _sc_scatter(n, eoh2, eom2, dsti2, z64, z16)

    gk = NUM_GRAPHS * K_SUPER
    s, r_acc, ent_acc = pl.pallas_call(
        _final1_body,
        grid=(n // bn,),
        in_specs=[pl.BlockSpec((bn, 64), lambda i: (i, 0)),
                  pl.BlockSpec((bn, 4), lambda i: (i, 0)),
                  pl.BlockSpec((bn, 1), lambda i: (i, 0)),
                  pl.BlockSpec((bn, 64), lambda i: (i, 0)),
                  pl.BlockSpec((bn, 64), lambda i: (i, 0)),
                  pl.BlockSpec((bn, 16), lambda i: (i, 0)),
                  pl.BlockSpec((bn, 16), lambda i: (i, 0)),
                  full((64, 64)), full((64, 64)), full((8, 64)),
                  full((64, 64)), full((64, K_SUPER)), full((8, K_SUPER))],
        out_specs=[pl.BlockSpec((bn, K_SUPER), lambda i: (i, 0)),
                   pl.BlockSpec((gk, 72), lambda i: (0, 0)),
                   pl.BlockSpec((1, 1), lambda i: (0, 0))],
        out_shape=[jax.ShapeDtypeStruct((n, K_SUPER), F32),
                   jax.ShapeDtypeStruct((gk, 72), F32),
                   jax.ShapeDtypeStruct((1, 1), F32)],
        name="gnn_final1",
    )(h1, xpv, bt, acch2[0], acch2[1], accm2[0], accm2[1],
      w1hx2, w1hm2, nvec2, w2h2,
      params["pool"]["W"],
      jnp.pad(params["pool"]["b"].reshape(1, K_SUPER), ((0, 7), (0, 0))))

    latf, muf, loss = pl.pallas_call(
        functools.partial(_final2_body, n),
        grid=(1,),
        in_specs=[full((gk, 72)), full((64, 64)), full((64, LATENT)),
                  full((8, 64)), full((8, LATENT)), full((1, 1))],
        out_specs=[full((gk, LATENT)), full((gk, 8)), full((1, 1))],
        out_shape=[jax.ShapeDtypeStruct((gk, LATENT), F32),
                   jax.ShapeDtypeStruct((gk, 8), F32),
                   jax.ShapeDtypeStruct((1, 1), F32)],
        name="gnn_final2",
    )(r_acc, params["out1"]["W"], params["out2"]["W"],
      jnp.pad(params["out1"]["b"].reshape(1, 64), ((0, 7), (0, 0))),
      jnp.pad(jnp.stack([params["out2"]["b"], params["latent_gain"]]),
              ((0, 6), (0, 0))),
      ent_acc)

    latent = latf.reshape(NUM_GRAPHS, K_SUPER, LATENT)
    mu = muf[:, 0:2].reshape(NUM_GRAPHS, K_SUPER, 2)
    return latent, s, loss.reshape(()), mu


# speedup vs baseline: 3.2817x; 1.1411x over previous
import functools

import jax
import jax.numpy as jnp
from jax import lax
from jax.experimental import pallas as pl
from jax.experimental.pallas import tpu as pltpu
from jax.experimental.pallas import tpu_sc as plsc

NUM_GRAPHS = 8
HID = 64
K_SUPER = 16
LATENT = 32

TW = 144
CE = 128
NSC = 2
NSUB = 16
NW = NSC * NSUB

F32 = jnp.float32


def _sp(x):
    return jnp.maximum(x, 0.0) + jnp.log1p(jnp.exp(-jnp.abs(x)))


def _lnk(x, g, b):
    m = jnp.mean(x, axis=-1, keepdims=True)
    v = jnp.mean((x - m) ** 2, axis=-1, keepdims=True)
    return (x - m) / jnp.sqrt(v + 1e-5) * g + b


def _lnk_mxu(x, g, b):
    u = jnp.full((x.shape[1], x.shape[1]), 1.0 / x.shape[1], F32)
    m = jnp.dot(x, u, preferred_element_type=F32)
    d = x - m
    v = jnp.dot(d * d, u, preferred_element_type=F32)
    return d * lax.rsqrt(v + 1e-5) * g + b


def _prep_body(xr, wd, ws, td, ts):
    xb = xr[...]
    pv = xb[:, 0:4]
    z = jnp.zeros((xb.shape[0], 12), F32)
    td[...] = jnp.concatenate(
        [jnp.dot(xb, wd[...], preferred_element_type=F32), -pv, z], axis=1)
    ts[...] = jnp.concatenate(
        [jnp.dot(xb, ws[...], preferred_element_type=F32), pv, z], axis=1)


def _edge_body(g, gw, vecp, bd, w3e, oh, om):
    gb = g[...]
    px = gb[:, 128:129]
    py = gb[:, 129:130]
    vx = gb[:, 130:131]
    vy = gb[:, 131:132]
    dist = px * px + py * py
    dvr = vx * px + vy * py
    r2 = jnp.minimum(1.0 / (dist + 0.05), 20.0)
    r6 = jnp.minimum(r2 * r2 * r2, 400.0)
    r12 = jnp.minimum(r6 * r6, 160000.0)
    zero = jnp.zeros_like(px)
    geo = jnp.concatenate([dist, dvr, r2, r6, r12, zero, zero, zero], axis=1)
    gpro = jnp.dot(geo, gw[...], preferred_element_type=F32)
    vp = vecp[...]
    ii = lax.broadcasted_iota(jnp.int32, (128, 128), 0) // 64
    jj = lax.broadcasted_iota(jnp.int32, (128, 128), 1) // 64
    u2 = jnp.where(ii == jj, 1.0 / 64, 0.0).astype(F32)
    ev = gb[:, 0:128] + gpro + vp[0:1, :]
    m = jnp.dot(ev, u2, preferred_element_type=F32)
    d = ev - m
    v = jnp.dot(d * d, u2, preferred_element_type=F32)
    ev = _sp(d * lax.rsqrt(v + 1e-5) * vp[1:2, :] + vp[2:3, :])
    t = jnp.dot(ev, bd[...], preferred_element_type=F32)
    e2 = _sp(t[:, 0:64] + vp[3:4, 0:64])
    sv = t[:, 64:65] + vp[3:4, 64:65]
    mh = jnp.dot(e2, w3e[...], preferred_element_type=F32) + vp[4:5, 0:64]
    oh[...] = mh
    om[...] = jnp.concatenate(
        [sv * px, sv * py, jnp.ones_like(px)] + [zero] * 13, axis=1)


def _node_update(xb, ah, am, w1hx, w1hm, nv, w2h):
    mvx = am[:, 0:1]
    mvy = am[:, 1:2]
    cnt = am[:, 2:3]
    den = jnp.maximum(cnt, 1.0)
    mh = ah / den
    mvx = mvx / den
    mvy = mvy / den
    mvn = jnp.sqrt((mvx + 1e-8) ** 2 + (mvy + 1e-8) ** 2)
    t = (jnp.dot(xb, w1hx[...], preferred_element_type=F32)
         + jnp.dot(mh, w1hm[...], preferred_element_type=F32)
         + mvn * nv[6:7, :] + nv[0:1, :])
    hh = _sp(_lnk(t, nv[1:2, :], nv[2:3, :]))
    return jnp.dot(hh, w2h[...], preferred_element_type=F32) + nv[3:4, :]


def _node1_body(xr, a0h, a1h, a0m, a1m, w1hx, w1hm, nvec, w2h, wsc,
                wd2, ws2, h1, t2d, t2s):
    xb = xr[...]
    ah = a0h[...] + a1h[...]
    am = a0m[...] + a1m[...]
    nv = nvec[...]
    upd = _node_update(xb, ah, am, w1hx, w1hm, nv, w2h)
    short = jnp.dot(xb, wsc[...], preferred_element_type=F32) + nv[7:8, :]
    h1b = _lnk(jnp.maximum(short + upd, 0.0), nv[4:5, :], nv[5:6, :])
    h1[...] = h1b
    pv = xb[:, 0:4]
    z = jnp.zeros((xb.shape[0], 12), F32)
    t2d[...] = jnp.concatenate(
        [jnp.dot(h1b, wd2[...], preferred_element_type=F32), -pv, z], axis=1)
    t2s[...] = jnp.concatenate(
        [jnp.dot(h1b, ws2[...], preferred_element_type=F32), pv, z], axis=1)


def _final1_body(h1r, xpvr, btr, a0h, a1h, a0m, a1m,
                 w1hx, w1hm, nvec, w2h, wp, fv16,
                 s_out, r_acc, ent_acc):
    i = pl.program_id(0)
    h1b = h1r[...]
    ah = a0h[...] + a1h[...]
    am = a0m[...] + a1m[...]
    nv = nvec[...]
    upd = _node_update(h1b, ah, am, w1hx, w1hm, nv, w2h)
    h2 = _lnk(jnp.maximum(h1b + upd, 0.0), nv[4:5, :], nv[5:6, :])

    lg = jnp.dot(h2, wp[...], preferred_element_type=F32) + fv16[0:1, :]
    mx = jnp.max(lg, axis=-1, keepdims=True)
    ex = jnp.exp(lg - mx)
    s = ex / jnp.sum(ex, axis=-1, keepdims=True)
    s_out[...] = s

    n = h1b.shape[0]
    ent = s * jnp.log(s + 1e-8)

    xpv = xpvr[...]
    one = jnp.ones((n, 1), F32)
    zed = jnp.zeros((n, 5), F32)
    hx = jnp.concatenate([h2, xpv[:, 0:1], xpv[:, 1:2], one, zed], axis=1)
    bt = btr[...]
    iota8 = lax.broadcasted_iota(jnp.int32, (n, NUM_GRAPHS), 1)
    oneh = (bt == iota8).astype(F32)
    rows = []
    for g in range(NUM_GRAPHS):
        sg = s * oneh[:, g:g + 1]
        rows.append(lax.dot_general(sg, hx, (((0,), (0,)), ((), ())),
                                    preferred_element_type=F32))
    r = jnp.concatenate(rows, axis=0)
    ent_s = jnp.sum(ent).reshape(1, 1)

    @pl.when(i == 0)
    def _():
        r_acc[...] = r
        ent_acc[...] = ent_s

    @pl.when(i > 0)
    def _():
        r_acc[...] = r_acc[...] + r
        ent_acc[...] = ent_acc[...] + ent_s


def _final2_body(n, rr, wo1, wo2, fv64, fv32, entr, latf, muf, loss):
    r = rr[...]
    den = r[:, 66:67] + 1e-8
    pooled = r[:, 0:64] / den
    mux = r[:, 64:65] / den
    muy = r[:, 65:66] / den
    z1 = jnp.maximum(jnp.dot(pooled, wo1[...], preferred_element_type=F32)
                     + fv64[0:1, :], 0.0)
    lat = (jnp.dot(z1, wo2[...], preferred_element_type=F32)
           + fv32[0:1, :]) * fv32[1:2, :]
    m = jnp.mean(lat, axis=-1, keepdims=True)
    v = jnp.mean((lat - m) ** 2, axis=-1, keepdims=True)
    latf[...] = (lat - m) / jnp.sqrt(v + 1e-5)
    z6 = jnp.zeros((r.shape[0], 6), F32)
    muf[...] = jnp.concatenate([mux, muy, z6], axis=1)
    loss[...] = -entr[...] / n


def _gather_sc_body(nch, ni, td, ts, dsti, srci, out,
                    di_v, si_v, rd_v, rs_v, sem1, sem2):
    cid = lax.axis_index("c")
    sid = lax.axis_index("s")
    wid = sid * NSC + cid

    def step(i, carry):
        c = i * NW + wid

        @pl.when(c < nch)
        def _():
            pltpu.sync_copy(dsti.at[c], di_v)
            pltpu.sync_copy(srci.at[c], si_v)
            cp1 = pltpu.async_copy(td.at[di_v], rd_v, sem1)
            cp2 = pltpu.async_copy(ts.at[si_v], rs_v, sem2)
            cp1.wait()
            cp2.wait()

            def addrow(r, carry2):
                for k in range(TW // 16):
                    rd_v[r, pl.ds(k * 16, 16)] = (
                        rd_v[r, pl.ds(k * 16, 16)] + rs_v[r, pl.ds(k * 16, 16)])
                return carry2

            lax.fori_loop(0, CE, addrow, 0)
            pltpu.sync_copy(rd_v, out.at[pl.ds(c * CE, CE)])

        return carry

    lax.fori_loop(0, ni, step, 0)


def _scatter_sc_body(n, nch, ni, eoh, eom, dsti, z64, z16, out_h, out_m,
                     acc_h, acc_m, di_v, rh_v, rm_v):
    cid = lax.axis_index("c")
    sid = lax.axis_index("s")
    wid = sid * NSC + cid
    rpt = n // NSUB

    pltpu.sync_copy(z64, acc_h.at[pl.ds(sid * rpt, rpt)])
    pltpu.sync_copy(z16, acc_m.at[pl.ds(sid * rpt, rpt)])
    plsc.subcore_barrier()

    def step(i, carry):
        c = i * NW + wid

        @pl.when(c < nch)
        def _():
            pltpu.sync_copy(dsti.at[c], di_v)
            pltpu.sync_copy(eoh.at[pl.ds(c * CE, CE)], rh_v)
            pltpu.sync_copy(eom.at[pl.ds(c * CE, CE)], rm_v)
            pltpu.sync_copy(rh_v, acc_h.at[di_v], add=True)
            pltpu.sync_copy(rm_v, acc_m.at[di_v], add=True)

        return carry

    lax.fori_loop(0, ni, step, 0)
    plsc.subcore_barrier()
    pltpu.sync_copy(acc_h.at[pl.ds(sid * rpt, rpt)],
                    out_h.at[cid, pl.ds(sid * rpt, rpt)])
    pltpu.sync_copy(acc_m.at[pl.ds(sid * rpt, rpt)],
                    out_m.at[cid, pl.ds(sid * rpt, rpt)])


def _sc_mesh():
    return plsc.VectorSubcoreMesh(core_axis_name="c", subcore_axis_name="s",
                                  num_cores=NSC, num_subcores=NSUB)


def _sc_gather(td, ts, dsti2, srci2):
    n = td.shape[0]
    nch = dsti2.shape[0]
    e = nch * CE
    ni = (nch + NW - 1) // NW
    call = pl.kernel(
        functools.partial(_gather_sc_body, nch, ni),
        out_type=jax.ShapeDtypeStruct((e, TW), F32),
        mesh=_sc_mesh(),
        scratch_types=[
            pltpu.VMEM((CE,), jnp.int32),
            pltpu.VMEM((CE,), jnp.int32),
            pltpu.VMEM((CE, TW), F32),
            pltpu.VMEM((CE, TW), F32),
            pltpu.SemaphoreType.DMA,
            pltpu.SemaphoreType.DMA,
        ],
        compiler_params=pltpu.CompilerParams(use_tc_tiling_on_sc=False),
        name="gnn_sc_gather",
    )
    del n
    return call(td, ts, dsti2, srci2)


def _sc_scatter(n, eoh, eom, dsti2, z64, z16):
    nch = dsti2.shape[0]
    ni = (nch + NW - 1) // NW
    call = pl.kernel(
        functools.partial(_scatter_sc_body, n, nch, ni),
        out_type=(jax.ShapeDtypeStruct((NSC, n, 64), F32),
                  jax.ShapeDtypeStruct((NSC, n, 16), F32)),
        mesh=_sc_mesh(),
        scratch_types=[
            pltpu.VMEM_SHARED((n, 64), F32),
            pltpu.VMEM_SHARED((n, 16), F32),
            pltpu.VMEM((CE,), jnp.int32),
            pltpu.VMEM((CE, 64), F32),
            pltpu.VMEM((CE, 16), F32),
        ],
        compiler_params=pltpu.CompilerParams(use_tc_tiling_on_sc=False),
        name="gnn_sc_scatter",
    )
    return call(eoh, eom, dsti2, z64, z16)


def _pack_edge(p, fin):
    w1e = p["phi_e"]["l1"]["W"]
    w1v = p["phi_v"]["l1"]["W"]
    wd = jnp.concatenate([w1e[:fin], w1v[:fin]], axis=1)
    ws = jnp.concatenate([w1e[fin:2 * fin], w1v[fin:2 * fin]], axis=1)
    gw = jnp.pad(jnp.concatenate([w1e[2 * fin:], w1v[2 * fin:]], axis=1),
                 ((0, 3), (0, 0)))
    z64 = jnp.zeros((HID,), F32)
    rows = [jnp.concatenate([p["phi_e"]["l1"]["b"], p["phi_v"]["l1"]["b"]]),
            jnp.concatenate([p["phi_e"]["g"], p["phi_v"]["g"]]),
            jnp.concatenate([p["phi_e"]["be"], p["phi_v"]["be"]]),
            jnp.concatenate([p["phi_e"]["l2"]["b"],
                             p["phi_v"]["l2"]["b"],
                             jnp.zeros((63,), F32)]),
            jnp.concatenate([p["phi_e"]["l3"]["b"], z64])]
    vecp = jnp.pad(jnp.stack(rows), ((0, 8 - len(rows)), (0, 0)))
    bd = jnp.zeros((128, 72), F32)
    bd = bd.at[0:64, 0:64].set(p["phi_e"]["l2"]["W"])
    bd = bd.at[64:128, 64:65].set(p["phi_v"]["l2"]["W"])
    return wd, ws, gw, vecp, bd, p["phi_e"]["l3"]["W"]


def _pack_node(p, ln, fin):
    w1h = p["phi_h"]["l1"]["W"]
    w1hx = w1h[:fin]
    w1hm = w1h[fin:fin + HID]
    w1hn = w1h[fin + HID]
    bsc = p["sc"]["b"] if "sc" in p else jnp.zeros((HID,), F32)
    rows = [p["phi_h"]["l1"]["b"], p["phi_h"]["g"], p["phi_h"]["be"],
            p["phi_h"]["l2"]["b"], ln["g"], ln["b"], w1hn, bsc]
    nvec = jnp.stack(rows)
    return w1hx, w1hm, nvec, p["phi_h"]["l2"]["W"]


def kernel(x, params, edge_index, batch):
    n, nf = x.shape
    e = edge_index.shape[1]
    assert e % CE == 0 and n % NSUB == 0
    nch = e // CE
    bn = 2000 if n % 2000 == 0 else n
    be = 2000 if e % 2000 == 0 else CE

    dsti2 = edge_index[1].reshape(nch, CE)
    srci2 = edge_index[0].reshape(nch, CE)
    z64 = jnp.zeros((n // NSUB, 64), F32)
    z16 = jnp.zeros((n // NSUB, 16), F32)
    xpv = x[:, 0:4]
    bt = batch.reshape(n, 1)

    p1 = params["gnn1"]
    p2 = params["gnn2"]
    wd1, ws1, gw1, vecp1, bd1, w3e1 = _pack_edge(p1, nf)
    wd2, ws2, gw2, vecp2, bd2, w3e2 = _pack_edge(p2, HID)
    w1hx1, w1hm1, nvec1, w2h1 = _pack_node(p1, params["ln1"], nf)
    w1hx2, w1hm2, nvec2, w2h2 = _pack_node(p2, params["ln2"], HID)
    wsc1 = p1["sc"]["W"]

    full = lambda shape: pl.BlockSpec(shape, lambda i: (0, 0))

    t1d, t1s = pl.pallas_call(
        _prep_body,
        grid=(n // bn,),
        in_specs=[pl.BlockSpec((bn, nf), lambda i: (i, 0)),
                  full((nf, 128)), full((nf, 128))],
        out_specs=[pl.BlockSpec((bn, TW), lambda i: (i, 0)),
                   pl.BlockSpec((bn, TW), lambda i: (i, 0))],
        out_shape=[jax.ShapeDtypeStruct((n, TW), F32),
                   jax.ShapeDtypeStruct((n, TW), F32)],
        name="gnn_prep",
    )(x, wd1, ws1)

    def edge_call(g, gw, vecp, bd, w3e):
        return pl.pallas_call(
            _edge_body,
            grid=(e // be,),
            in_specs=[pl.BlockSpec((be, TW), lambda i: (i, 0)),
                      full((8, 128)), full((8, 128)), full((128, 72)),
                      full((64, 64))],
            out_specs=[pl.BlockSpec((be, 64), lambda i: (i, 0)),
                       pl.BlockSpec((be, 16), lambda i: (i, 0))],
            out_shape=[jax.ShapeDtypeStruct((e, 64), F32),
                       jax.ShapeDtypeStruct((e, 16), F32)],
            name="gnn_edge_mlp",
        )(g, gw, vecp, bd, w3e)

    g1 = _sc_gather(t1d, t1s, dsti2, srci2)
    eoh1, eom1 = edge_call(g1, gw1, vecp1, bd1, w3e1)
    acch1, accm1 = _sc_scatter(n, eoh1, eom1, dsti2, z64, z16)

    h1, t2d, t2s = pl.pallas_call(
        _node1_body,
        grid=(n // bn,),
        in_specs=[pl.BlockSpec((bn, nf), lambda i: (i, 0)),
                  pl.BlockSpec((bn, 64), lambda i: (i, 0)),
                  pl.BlockSpec((bn, 64), lambda i: (i, 0)),
                  pl.BlockSpec((bn, 16), lambda i: (i, 0)),
                  pl.BlockSpec((bn, 16), lambda i: (i, 0)),
                  full((nf, 64)), full((64, 64)), full((8, 64)),
                  full((64, 64)), full((nf, 64)),
                  full((64, 128)), full((64, 128))],
        out_specs=[pl.BlockSpec((bn, 64), lambda i: (i, 0)),
                   pl.BlockSpec((bn, TW), lambda i: (i, 0)),
                   pl.BlockSpec((bn, TW), lambda i: (i, 0))],
        out_shape=[jax.ShapeDtypeStruct((n, 64), F32),
                   jax.ShapeDtypeStruct((n, TW), F32),
                   jax.ShapeDtypeStruct((n, TW), F32)],
        name="gnn_node1",
    )(x, acch1[0], acch1[1], accm1[0], accm1[1],
      w1hx1, w1hm1, nvec1, w2h1, wsc1, wd2, ws2)

    g2 = _sc_gather(t2d, t2s, dsti2, srci2)
    eoh2, eom2 = edge_call(g2, gw2, vecp2, bd2, w3e2)
    acch2, accm2 = _sc_scatter(n, eoh2, eom2, dsti2, z64, z16)

    gk = NUM_GRAPHS * K_SUPER
    s, r_acc, ent_acc = pl.pallas_call(
        _final1_body,
        grid=(n // bn,),
        in_specs=[pl.BlockSpec((bn, 64), lambda i: (i, 0)),
                  pl.BlockSpec((bn, 4), lambda i: (i, 0)),
                  pl.BlockSpec((bn, 1), lambda i: (i, 0)),
                  pl.BlockSpec((bn, 64), lambda i: (i, 0)),
                  pl.BlockSpec((bn, 64), lambda i: (i, 0)),
                  pl.BlockSpec((bn, 16), lambda i: (i, 0)),
                  pl.BlockSpec((bn, 16), lambda i: (i, 0)),
                  full((64, 64)), full((64, 64)), full((8, 64)),
                  full((64, 64)), full((64, K_SUPER)), full((8, K_SUPER))],
        out_specs=[pl.BlockSpec((bn, K_SUPER), lambda i: (i, 0)),
                   pl.BlockSpec((gk, 72), lambda i: (0, 0)),
                   pl.BlockSpec((1, 1), lambda i: (0, 0))],
        out_shape=[jax.ShapeDtypeStruct((n, K_SUPER), F32),
                   jax.ShapeDtypeStruct((gk, 72), F32),
                   jax.ShapeDtypeStruct((1, 1), F32)],
        name="gnn_final1",
    )(h1, xpv, bt, acch2[0], acch2[1], accm2[0], accm2[1],
      w1hx2, w1hm2, nvec2, w2h2,
      params["pool"]["W"],
      jnp.pad(params["pool"]["b"].reshape(1, K_SUPER), ((0, 7), (0, 0))))

    latf, muf, loss = pl.pallas_call(
        functools.partial(_final2_body, n),
        grid=(1,),
        in_specs=[full((gk, 72)), full((64, 64)), full((64, LATENT)),
                  full((8, 64)), full((8, LATENT)), full((1, 1))],
        out_specs=[full((gk, LATENT)), full((gk, 8)), full((1, 1))],
        out_shape=[jax.ShapeDtypeStruct((gk, LATENT), F32),
                   jax.ShapeDtypeStruct((gk, 8), F32),
                   jax.ShapeDtypeStruct((1, 1), F32)],
        name="gnn_final2",
    )(r_acc, params["out1"]["W"], params["out2"]["W"],
      jnp.pad(params["out1"]["b"].reshape(1, 64), ((0, 7), (0, 0))),
      jnp.pad(jnp.stack([params["out2"]["b"], params["latent_gain"]]),
              ((0, 6), (0, 0))),
      ent_acc)

    latent = latf.reshape(NUM_GRAPHS, K_SUPER, LATENT)
    mu = muf[:, 0:2].reshape(NUM_GRAPHS, K_SUPER, 2)
    return latent, s, loss.reshape(()), mu


# speedup vs baseline: 5.6523x; 1.7224x over previous
import functools

import jax
import jax.numpy as jnp
from jax import lax
from jax.experimental import pallas as pl
from jax.experimental.pallas import tpu as pltpu
from jax.experimental.pallas import tpu_sc as plsc

NUM_GRAPHS = 8
HID = 64
K_SUPER = 16
LATENT = 32

TW = 144
CE = 128
NSC = 2
NSUB = 16
NW = NSC * NSUB

F32 = jnp.float32


def _sp(x):
    return jnp.maximum(x, 0.0) + jnp.log1p(jnp.exp(-jnp.abs(x)))


def _lnk(x, g, b):
    m = jnp.mean(x, axis=-1, keepdims=True)
    v = jnp.mean((x - m) ** 2, axis=-1, keepdims=True)
    return (x - m) / jnp.sqrt(v + 1e-5) * g + b


def _lnk_mxu(x, g, b):
    u = jnp.full((x.shape[1], x.shape[1]), 1.0 / x.shape[1], F32)
    m = jnp.dot(x, u, preferred_element_type=F32)
    d = x - m
    v = jnp.dot(d * d, u, preferred_element_type=F32)
    return d * lax.rsqrt(v + 1e-5) * g + b


def _prep_body(xr, wd, ws, tmd, tms, ttd, tts):
    xb = xr[...]
    pv = xb[:, 0:4]
    z = jnp.zeros((xb.shape[0], 12), F32)
    tmd[...] = jnp.dot(xb, wd[...], preferred_element_type=F32)
    tms[...] = jnp.dot(xb, ws[...], preferred_element_type=F32)
    ttd[...] = jnp.concatenate([-pv, z], axis=1)
    tts[...] = jnp.concatenate([pv, z], axis=1)


def _cmat(rows, cols, fn):
    ri = lax.broadcasted_iota(jnp.int32, (rows, cols), 0)
    ci = lax.broadcasted_iota(jnp.int32, (rows, cols), 1)
    return jnp.where(fn(ri, ci), 1.0, 0.0).astype(F32)


def _geo_body(gt, o8):
    t = gt[...]
    n = t.shape[0]
    sh = _cmat(16, 16, lambda r, c: (r == c + 2) & (c < 2))
    ts2 = jnp.dot(t, sh, preferred_element_type=F32)
    sq = t * t
    pr = t * ts2
    m1 = _cmat(16, 8, lambda r, c: (r < 2) & (c == 0))
    m2 = _cmat(16, 8, lambda r, c: (r < 2) & (c == 1))
    m3 = _cmat(16, 8, lambda r, c: (r < 2) & (c == r + 5))
    s = (jnp.dot(sq, m1, preferred_element_type=F32)
         + jnp.dot(pr, m2, preferred_element_type=F32)
         + jnp.dot(t, m3, preferred_element_type=F32))
    li = lax.broadcasted_iota(jnp.int32, (n, 8), 1)
    s = jnp.where(li == 7, 1.0, s)
    e0 = _cmat(8, 8, lambda r, c: r == 0)
    dd = jnp.dot(s, e0, preferred_element_type=F32)
    r2 = jnp.minimum(1.0 / (dd + 0.05), 20.0)
    r6 = jnp.minimum(r2 * r2 * r2, 400.0)
    r12 = jnp.minimum(r6 * r6, 160000.0)
    o8[...] = jnp.where(li == 2, r2,
                        jnp.where(li == 3, r6,
                                  jnp.where(li == 4, r12, s)))


def _edge_body(gm, g8, gw, vecp, bd, w3e, oh, om):
    ev0 = gm[...]
    geo = g8[...]
    gpro = jnp.dot(geo, gw[...], preferred_element_type=F32)
    vp = vecp[...]
    ii = lax.broadcasted_iota(jnp.int32, (128, 128), 0) // 64
    jj = lax.broadcasted_iota(jnp.int32, (128, 128), 1) // 64
    u2 = jnp.where(ii == jj, 1.0 / 64, 0.0).astype(F32)
    ev = ev0 + gpro + vp[0:1, :]
    m = jnp.dot(ev, u2, preferred_element_type=F32)
    d = ev - m
    v = jnp.dot(d * d, u2, preferred_element_type=F32)
    ev = _sp(d * lax.rsqrt(v + 1e-5) * vp[1:2, :] + vp[2:3, :])
    t = jnp.dot(ev, bd[...], preferred_element_type=F32)
    e2 = _sp(t[:, 0:64] + vp[3:4, 0:64])
    sv = t[:, 64:65] + vp[3:4, 64:65]
    mh = jnp.dot(e2, w3e[...], preferred_element_type=F32) + vp[4:5, 0:64]
    oh[...] = mh
    sela = _cmat(8, 16, lambda r, c: (c < 2) & (r == c + 5))
    selb = _cmat(8, 16, lambda r, c: (r == 7) & (c == 2))
    om[...] = (jnp.dot(sv * geo, sela, preferred_element_type=F32)
               + jnp.dot(geo, selb, preferred_element_type=F32))


def _node_update(xb, ah, am, w1hx, w1hm, nv, w2h):
    mvx = am[:, 0:1]
    mvy = am[:, 1:2]
    cnt = am[:, 2:3]
    den = jnp.maximum(cnt, 1.0)
    mh = ah / den
    mvx = mvx / den
    mvy = mvy / den
    mvn = jnp.sqrt((mvx + 1e-8) ** 2 + (mvy + 1e-8) ** 2)
    t = (jnp.dot(xb, w1hx[...], preferred_element_type=F32)
         + jnp.dot(mh, w1hm[...], preferred_element_type=F32)
         + mvn * nv[6:7, :] + nv[0:1, :])
    hh = _sp(_lnk(t, nv[1:2, :], nv[2:3, :]))
    return jnp.dot(hh, w2h[...], preferred_element_type=F32) + nv[3:4, :]


def _node1_body(xr, a0h, a1h, a0m, a1m, w1hx, w1hm, nvec, w2h, wsc,
                wd2, ws2, h1, t2d, t2s):
    xb = xr[...]
    ah = a0h[...] + a1h[...]
    am = a0m[...] + a1m[...]
    nv = nvec[...]
    upd = _node_update(xb, ah, am, w1hx, w1hm, nv, w2h)
    short = jnp.dot(xb, wsc[...], preferred_element_type=F32) + nv[7:8, :]
    h1b = _lnk(jnp.maximum(short + upd, 0.0), nv[4:5, :], nv[5:6, :])
    h1[...] = h1b
    t2d[...] = jnp.dot(h1b, wd2[...], preferred_element_type=F32)
    t2s[...] = jnp.dot(h1b, ws2[...], preferred_element_type=F32)


def _final1_body(h1r, xpvr, btr, a0h, a1h, a0m, a1m,
                 w1hx, w1hm, nvec, w2h, wp, fv16,
                 s_out, r_acc, ent_acc):
    i = pl.program_id(0)
    h1b = h1r[...]
    ah = a0h[...] + a1h[...]
    am = a0m[...] + a1m[...]
    nv = nvec[...]
    upd = _node_update(h1b, ah, am, w1hx, w1hm, nv, w2h)
    h2 = _lnk(jnp.maximum(h1b + upd, 0.0), nv[4:5, :], nv[5:6, :])

    lg = jnp.dot(h2, wp[...], preferred_element_type=F32) + fv16[0:1, :]
    mx = jnp.max(lg, axis=-1, keepdims=True)
    ex = jnp.exp(lg - mx)
    s = ex / jnp.sum(ex, axis=-1, keepdims=True)
    s_out[...] = s

    n = h1b.shape[0]
    ent = s * jnp.log(s + 1e-8)

    xpv = xpvr[...]
    one = jnp.ones((n, 1), F32)
    zed = jnp.zeros((n, 5), F32)
    hx = jnp.concatenate([h2, xpv[:, 0:1], xpv[:, 1:2], one, zed], axis=1)
    bt = btr[...]
    iota8 = lax.broadcasted_iota(jnp.int32, (n, NUM_GRAPHS), 1)
    oneh = (bt == iota8).astype(F32)
    rows = []
    for g in range(NUM_GRAPHS):
        sg = s * oneh[:, g:g + 1]
        rows.append(lax.dot_general(sg, hx, (((0,), (0,)), ((), ())),
                                    preferred_element_type=F32))
    r = jnp.concatenate(rows, axis=0)
    ent_s = jnp.sum(ent).reshape(1, 1)

    @pl.when(i == 0)
    def _():
        r_acc[...] = r
        ent_acc[...] = ent_s

    @pl.when(i > 0)
    def _():
        r_acc[...] = r_acc[...] + r
        ent_acc[...] = ent_acc[...] + ent_s


def _final2_body(n, rr, wo1, wo2, fv64, fv32, entr, latf, muf, loss):
    r = rr[...]
    den = r[:, 66:67] + 1e-8
    pooled = r[:, 0:64] / den
    mux = r[:, 64:65] / den
    muy = r[:, 65:66] / den
    z1 = jnp.maximum(jnp.dot(pooled, wo1[...], preferred_element_type=F32)
                     + fv64[0:1, :], 0.0)
    lat = (jnp.dot(z1, wo2[...], preferred_element_type=F32)
           + fv32[0:1, :]) * fv32[1:2, :]
    m = jnp.mean(lat, axis=-1, keepdims=True)
    v = jnp.mean((lat - m) ** 2, axis=-1, keepdims=True)
    latf[...] = (lat - m) / jnp.sqrt(v + 1e-5)
    z6 = jnp.zeros((r.shape[0], 6), F32)
    muf[...] = jnp.concatenate([mux, muy, z6], axis=1)
    loss[...] = -entr[...] / n


def _gather_sc_body(nch, ni, tails, *args):
    if tails:
        (td, ts, tdt, tst, dsti, srci, outm, outt,
         di, si, rdm, rsm, rdt, rst, gs0, gs1, os0, os1) = args
    else:
        (td, ts, dsti, srci, outm,
         di, si, rdm, rsm, gs0, gs1, os0, os1) = args
    cid = lax.axis_index("c")
    sid = lax.axis_index("s")
    wid = sid * NSC + cid
    gsems = (gs0, gs1)
    osems = (os0, os1)

    def chunk_id(i):
        return i * NW + wid

    def issue(i, b):
        c = chunk_id(i)

        @pl.when(c < nch)
        def _():
            pltpu.sync_copy(dsti.at[c], di.at[b])
            pltpu.sync_copy(srci.at[c], si.at[b])
            pltpu.async_copy(td.at[di.at[b]], rdm.at[b], gsems[b])
            pltpu.async_copy(ts.at[si.at[b]], rsm.at[b], gsems[b])
            if tails:
                pltpu.async_copy(tdt.at[di.at[b]], rdt.at[b], gsems[b])
                pltpu.async_copy(tst.at[si.at[b]], rst.at[b], gsems[b])

    def reclaim(i, b):
        @pl.when((i >= 0) & (chunk_id(i) < nch))
        def _():
            pltpu.make_async_copy(rdm.at[b],
                                  outm.at[pl.ds(chunk_id(i) * CE, CE)],
                                  osems[b]).wait()
            if tails:
                pltpu.make_async_copy(rdt.at[b],
                                      outt.at[pl.ds(chunk_id(i) * CE, CE)],
                                      osems[b]).wait()

    issue(0, 0)

    def pair(j, carry):
        for b in range(2):
            i = 2 * j + b
            c = chunk_id(i)
            reclaim(i - 1, 1 - b)
            issue(i + 1, 1 - b)

            @pl.when(c < nch)
            def _():
                pltpu.make_async_copy(td.at[di.at[b]], rdm.at[b],
                                      gsems[b]).wait()
                pltpu.make_async_copy(ts.at[si.at[b]], rsm.at[b],
                                      gsems[b]).wait()
                if tails:
                    pltpu.make_async_copy(tdt.at[di.at[b]], rdt.at[b],
                                          gsems[b]).wait()
                    pltpu.make_async_copy(tst.at[si.at[b]], rst.at[b],
                                          gsems[b]).wait()

                def addrow(r, carry2):
                    for k in range(128 // 16):
                        rdm[b, r, pl.ds(k * 16, 16)] = (
                            rdm[b, r, pl.ds(k * 16, 16)]
                            + rsm[b, r, pl.ds(k * 16, 16)])
                    if tails:
                        rdt[b, r, pl.ds(0, 16)] = (
                            rdt[b, r, pl.ds(0, 16)]
                            + rst[b, r, pl.ds(0, 16)])
                    return carry2

                lax.fori_loop(0, CE, addrow, 0)
                pltpu.async_copy(rdm.at[b], outm.at[pl.ds(c * CE, CE)],
                                 osems[b])
                if tails:
                    pltpu.async_copy(rdt.at[b], outt.at[pl.ds(c * CE, CE)],
                                     osems[b])

        return carry

    npairs = (ni + 1) // 2
    lax.fori_loop(0, npairs, pair, 0)

    last = 2 * npairs - 1
    reclaim(last, last % 2)


def _scatter_sc_body(n, nch, ni, eoh, eom, dsti, inih, inim, out_h, out_m,
                     acc_h, acc_m, di, rh, rm, ls0, ls1):
    cid = lax.axis_index("c")
    sid = lax.axis_index("s")
    wid = sid * NSC + cid
    rpt = n // NSUB
    lsems = (ls0, ls1)

    pltpu.sync_copy(inih.at[cid, pl.ds(sid * rpt, rpt)],
                    acc_h.at[pl.ds(sid * rpt, rpt)])
    pltpu.sync_copy(inim.at[cid, pl.ds(sid * rpt, rpt)],
                    acc_m.at[pl.ds(sid * rpt, rpt)])
    plsc.subcore_barrier()

    def chunk_id(i):
        return i * NW + wid

    def issue(i, b):
        c = chunk_id(i)

        @pl.when(c < nch)
        def _():
            pltpu.sync_copy(dsti.at[c], di.at[b])
            pltpu.async_copy(eoh.at[pl.ds(c * CE, CE)], rh.at[b], lsems[b])
            pltpu.async_copy(eom.at[pl.ds(c * CE, CE)], rm.at[b], lsems[b])

    issue(0, 0)

    def pair(j, carry):
        for b in range(2):
            i = 2 * j + b
            c = chunk_id(i)
            issue(i + 1, 1 - b)

            @pl.when(c < nch)
            def _():
                pltpu.make_async_copy(eoh.at[pl.ds(c * CE, CE)], rh.at[b],
                                      lsems[b]).wait()
                pltpu.make_async_copy(eom.at[pl.ds(c * CE, CE)], rm.at[b],
                                      lsems[b]).wait()
                pltpu.sync_copy(rh.at[b], acc_h.at[di.at[b]], add=True)
                pltpu.sync_copy(rm.at[b], acc_m.at[di.at[b]], add=True)

        return carry

    lax.fori_loop(0, (ni + 1) // 2, pair, 0)
    plsc.subcore_barrier()
    pltpu.sync_copy(acc_h.at[pl.ds(sid * rpt, rpt)],
                    out_h.at[cid, pl.ds(sid * rpt, rpt)])
    pltpu.sync_copy(acc_m.at[pl.ds(sid * rpt, rpt)],
                    out_m.at[cid, pl.ds(sid * rpt, rpt)])


def _sc_mesh():
    return plsc.VectorSubcoreMesh(core_axis_name="c", subcore_axis_name="s",
                                  num_cores=NSC, num_subcores=NSUB)


def _sc_gather1(td, ts, tdt, tst, dsti2, srci2):
    nch = dsti2.shape[0]
    e = nch * CE
    ni = (nch + NW - 1) // NW
    call = pl.kernel(
        functools.partial(_gather_sc_body, nch, ni, True),
        out_type=(jax.ShapeDtypeStruct((e, 128), F32),
                  jax.ShapeDtypeStruct((e, 16), F32)),
        mesh=_sc_mesh(),
        scratch_types=[
            pltpu.VMEM((2, CE), jnp.int32),
            pltpu.VMEM((2, CE), jnp.int32),
            pltpu.VMEM((2, CE, 128), F32),
            pltpu.VMEM((2, CE, 128), F32),
            pltpu.VMEM((2, CE, 16), F32),
            pltpu.VMEM((2, CE, 16), F32),
            pltpu.SemaphoreType.DMA,
            pltpu.SemaphoreType.DMA,
            pltpu.SemaphoreType.DMA,
            pltpu.SemaphoreType.DMA,
        ],
        compiler_params=pltpu.CompilerParams(use_tc_tiling_on_sc=False),
        name="gnn_sc_gather1",
    )
    return call(td, ts, tdt, tst, dsti2, srci2)


def _sc_gather2(td, ts, dsti2, srci2):
    nch = dsti2.shape[0]
    e = nch * CE
    ni = (nch + NW - 1) // NW
    call = pl.kernel(
        functools.partial(_gather_sc_body, nch, ni, False),
        out_type=jax.ShapeDtypeStruct((e, 128), F32),
        mesh=_sc_mesh(),
        scratch_types=[
            pltpu.VMEM((2, CE), jnp.int32),
            pltpu.VMEM((2, CE), jnp.int32),
            pltpu.VMEM((2, CE, 128), F32),
            pltpu.VMEM((2, CE, 128), F32),
            pltpu.SemaphoreType.DMA,
            pltpu.SemaphoreType.DMA,
            pltpu.SemaphoreType.DMA,
            pltpu.SemaphoreType.DMA,
        ],
        compiler_params=pltpu.CompilerParams(use_tc_tiling_on_sc=False),
        name="gnn_sc_gather2",
    )
    return call(td, ts, dsti2, srci2)


def _sc_scatter(n, eoh, eom, dsti2, inih, inim):
    nch = dsti2.shape[0]
    ni = (nch + NW - 1) // NW
    call = pl.kernel(
        functools.partial(_scatter_sc_body, n, nch, ni),
        out_type=(jax.ShapeDtypeStruct((NSC, n, 64), F32),
                  jax.ShapeDtypeStruct((NSC, n, 16), F32)),
        mesh=_sc_mesh(),
        scratch_types=[
            pltpu.VMEM_SHARED((n, 64), F32),
            pltpu.VMEM_SHARED((n, 16), F32),
            pltpu.VMEM((2, CE), jnp.int32),
            pltpu.VMEM((2, CE, 64), F32),
            pltpu.VMEM((2, CE, 16), F32),
            pltpu.SemaphoreType.DMA,
            pltpu.SemaphoreType.DMA,
        ],
        compiler_params=pltpu.CompilerParams(use_tc_tiling_on_sc=False),
        name="gnn_sc_scatter",
    )
    return call(eoh, eom, dsti2, inih, inim)


def _pack_edge(p, fin):
    w1e = p["phi_e"]["l1"]["W"]
    w1v = p["phi_v"]["l1"]["W"]
    wd = jnp.concatenate([w1e[:fin], w1v[:fin]], axis=1)
    ws = jnp.concatenate([w1e[fin:2 * fin], w1v[fin:2 * fin]], axis=1)
    gw = jnp.pad(jnp.concatenate([w1e[2 * fin:], w1v[2 * fin:]], axis=1),
                 ((0, 3), (0, 0)))
    z64 = jnp.zeros((HID,), F32)
    rows = [jnp.concatenate([p["phi_e"]["l1"]["b"], p["phi_v"]["l1"]["b"]]),
            jnp.concatenate([p["phi_e"]["g"], p["phi_v"]["g"]]),
            jnp.concatenate([p["phi_e"]["be"], p["phi_v"]["be"]]),
            jnp.concatenate([p["phi_e"]["l2"]["b"],
                             p["phi_v"]["l2"]["b"],
                             jnp.zeros((63,), F32)]),
            jnp.concatenate([p["phi_e"]["l3"]["b"], z64])]
    vecp = jnp.pad(jnp.stack(rows), ((0, 8 - len(rows)), (0, 0)))
    bd = jnp.zeros((128, 72), F32)
    bd = bd.at[0:64, 0:64].set(p["phi_e"]["l2"]["W"])
    bd = bd.at[64:128, 64:65].set(p["phi_v"]["l2"]["W"])
    return wd, ws, gw, vecp, bd, p["phi_e"]["l3"]["W"]


def _pack_node(p, ln, fin):
    w1h = p["phi_h"]["l1"]["W"]
    w1hx = w1h[:fin]
    w1hm = w1h[fin:fin + HID]
    w1hn = w1h[fin + HID]
    bsc = p["sc"]["b"] if "sc" in p else jnp.zeros((HID,), F32)
    rows = [p["phi_h"]["l1"]["b"], p["phi_h"]["g"], p["phi_h"]["be"],
            p["phi_h"]["l2"]["b"], ln["g"], ln["b"], w1hn, bsc]
    nvec = jnp.stack(rows)
    return w1hx, w1hm, nvec, p["phi_h"]["l2"]["W"]


def kernel(x, params, edge_index, batch):
    n, nf = x.shape
    e = edge_index.shape[1]
    assert e % CE == 0 and n % NSUB == 0
    nch = e // CE
    bn = 2000 if n % 2000 == 0 else n
    be = 2000 if e % 2000 == 0 else CE

    dsti2 = edge_index[1].reshape(nch, CE)
    srci2 = edge_index[0].reshape(nch, CE)
    xpv = x[:, 0:4]
    bt = batch.reshape(n, 1)

    p1 = params["gnn1"]
    p2 = params["gnn2"]
    wd1, ws1, gw1, vecp1, bd1, w3e1 = _pack_edge(p1, nf)
    wd2, ws2, gw2, vecp2, bd2, w3e2 = _pack_edge(p2, HID)
    w1hx1, w1hm1, nvec1, w2h1 = _pack_node(p1, params["ln1"], nf)
    w1hx2, w1hm2, nvec2, w2h2 = _pack_node(p2, params["ln2"], HID)
    wsc1 = p1["sc"]["W"]

    full = lambda shape: pl.BlockSpec(shape, lambda i: (0, 0))

    t1d, t1s, tt1d, tt1s = pl.pallas_call(
        _prep_body,
        grid=(n // bn,),
        in_specs=[pl.BlockSpec((bn, nf), lambda i: (i, 0)),
                  full((nf, 128)), full((nf, 128))],
        out_specs=[pl.BlockSpec((bn, 128), lambda i: (i, 0)),
                   pl.BlockSpec((bn, 128), lambda i: (i, 0)),
                   pl.BlockSpec((bn, 16), lambda i: (i, 0)),
                   pl.BlockSpec((bn, 16), lambda i: (i, 0))],
        out_shape=[jax.ShapeDtypeStruct((n, 128), F32),
                   jax.ShapeDtypeStruct((n, 128), F32),
                   jax.ShapeDtypeStruct((n, 16), F32),
                   jax.ShapeDtypeStruct((n, 16), F32)],
        name="gnn_prep",
    )(x, wd1, ws1)

    def edge_call(gm, g8, gw, vecp, bd, w3e):
        es = gm.shape[0]
        return pl.pallas_call(
            _edge_body,
            grid=(es // be,),
            in_specs=[pl.BlockSpec((be, 128), lambda i: (i, 0)),
                      pl.BlockSpec((be, 8), lambda i: (i, 0)),
                      full((8, 128)), full((8, 128)), full((128, 72)),
                      full((64, 64))],
            out_specs=[pl.BlockSpec((be, 64), lambda i: (i, 0)),
                       pl.BlockSpec((be, 16), lambda i: (i, 0))],
            out_shape=[jax.ShapeDtypeStruct((es, 64), F32),
                       jax.ShapeDtypeStruct((es, 16), F32)],
            name="gnn_edge_mlp",
        )(gm, g8, gw, vecp, bd, w3e)

    def geo_call(gt):
        es = gt.shape[0]
        return pl.pallas_call(
            _geo_body,
            grid=(es // be,),
            in_specs=[pl.BlockSpec((be, 16), lambda i: (i, 0))],
            out_specs=pl.BlockSpec((be, 8), lambda i: (i, 0)),
            out_shape=jax.ShapeDtypeStruct((es, 8), F32),
            name="gnn_geo",
        )(gt)

    nsl = 2
    sch = nch // nsl
    dsls = [dsti2[sl * sch:(sl + 1) * sch] for sl in range(nsl)]
    ssls = [srci2[sl * sch:(sl + 1) * sch] for sl in range(nsl)]
    acch1 = jnp.zeros((NSC, n, 64), F32)
    accm1 = jnp.zeros((NSC, n, 16), F32)
    geo8s = []
    for sl in range(nsl):
        g1m, g1t = _sc_gather1(t1d, t1s, tt1d, tt1s, dsls[sl], ssls[sl])
        geo8 = geo_call(g1t)
        geo8s.append(geo8)
        eoh1, eom1 = edge_call(g1m, geo8, gw1, vecp1, bd1, w3e1)
        acch1, accm1 = _sc_scatter(n, eoh1, eom1, dsls[sl], acch1, accm1)

    h1, t2d, t2s = pl.pallas_call(
        _node1_body,
        grid=(n // bn,),
        in_specs=[pl.BlockSpec((bn, nf), lambda i: (i, 0)),
                  pl.BlockSpec((bn, 64), lambda i: (i, 0)),
                  pl.BlockSpec((bn, 64), lambda i: (i, 0)),
                  pl.BlockSpec((bn, 16), lambda i: (i, 0)),
                  pl.BlockSpec((bn, 16), lambda i: (i, 0)),
                  full((nf, 64)), full((64, 64)), full((8, 64)),
                  full((64, 64)), full((nf, 64)),
                  full((64, 128)), full((64, 128))],
        out_specs=[pl.BlockSpec((bn, 64), lambda i: (i, 0)),
                   pl.BlockSpec((bn, 128), lambda i: (i, 0)),
                   pl.BlockSpec((bn, 128), lambda i: (i, 0))],
        out_shape=[jax.ShapeDtypeStruct((n, 64), F32),
                   jax.ShapeDtypeStruct((n, 128), F32),
                   jax.ShapeDtypeStruct((n, 128), F32)],
        name="gnn_node1",
    )(x, acch1[0], acch1[1], accm1[0], accm1[1],
      w1hx1, w1hm1, nvec1, w2h1, wsc1, wd2, ws2)

    acch2 = jnp.zeros((NSC, n, 64), F32)
    accm2 = jnp.zeros((NSC, n, 16), F32)
    for sl in range(nsl):
        g2m = _sc_gather2(t2d, t2s, dsls[sl], ssls[sl])
        eoh2, eom2 = edge_call(g2m, geo8s[sl], gw2, vecp2, bd2, w3e2)
        acch2, accm2 = _sc_scatter(n, eoh2, eom2, dsls[sl], acch2, accm2)

    gk = NUM_GRAPHS * K_SUPER
    s, r_acc, ent_acc = pl.pallas_call(
        _final1_body,
        grid=(n // bn,),
        in_specs=[pl.BlockSpec((bn, 64), lambda i: (i, 0)),
                  pl.BlockSpec((bn, 4), lambda i: (i, 0)),
                  pl.BlockSpec((bn, 1), lambda i: (i, 0)),
                  pl.BlockSpec((bn, 64), lambda i: (i, 0)),
                  pl.BlockSpec((bn, 64), lambda i: (i, 0)),
                  pl.BlockSpec((bn, 16), lambda i: (i, 0)),
                  pl.BlockSpec((bn, 16), lambda i: (i, 0)),
                  full((64, 64)), full((64, 64)), full((8, 64)),
                  full((64, 64)), full((64, K_SUPER)), full((8, K_SUPER))],
        out_specs=[pl.BlockSpec((bn, K_SUPER), lambda i: (i, 0)),
                   pl.BlockSpec((gk, 72), lambda i: (0, 0)),
                   pl.BlockSpec((1, 1), lambda i: (0, 0))],
        out_shape=[jax.ShapeDtypeStruct((n, K_SUPER), F32),
                   jax.ShapeDtypeStruct((gk, 72), F32),
                   jax.ShapeDtypeStruct((1, 1), F32)],
        name="gnn_final1",
    )(h1, xpv, bt, acch2[0], acch2[1], accm2[0], accm2[1],
      w1hx2, w1hm2, nvec2, w2h2,
      params["pool"]["W"],
      jnp.pad(params["pool"]["b"].reshape(1, K_SUPER), ((0, 7), (0, 0))))

    latf, muf, loss = pl.pallas_call(
        functools.partial(_final2_body, n),
        grid=(1,),
        in_specs=[full((gk, 72)), full((64, 64)), full((64, LATENT)),
                  full((8, 64)), full((8, LATENT)), full((1, 1))],
        out_specs=[full((gk, LATENT)), full((gk, 8)), full((1, 1))],
        out_shape=[jax.ShapeDtypeStruct((gk, LATENT), F32),
                   jax.ShapeDtypeStruct((gk, 8), F32),
                   jax.ShapeDtypeStruct((1, 1), F32)],
        name="gnn_final2",
    )(r_acc, params["out1"]["W"], params["out2"]["W"],
      jnp.pad(params["out1"]["b"].reshape(1, 64), ((0, 7), (0, 0))),
      jnp.pad(jnp.stack([params["out2"]["b"], params["latent_gain"]]),
              ((0, 6), (0, 0))),
      ent_acc)

    latent = latf.reshape(NUM_GRAPHS, K_SUPER, LATENT)
    mu = muf[:, 0:2].reshape(NUM_GRAPHS, K_SUPER, 2)
    return latent, s, loss.reshape(()), mu
